# baseline (speedup 1.0000x reference)
import jax
import jax.numpy as jnp
from jax.experimental import pallas as pl
from jax.experimental.pallas import tpu as pltpu

_DROP_RATIO = 0.2
_B_COEF = 1.0
_C_COEF = 1.0
_EPS = 1e-08

_R = 8
_NB = 8
_LA = 6


def _pass1_kernel(nblk, x_any, copy_any, sums_ref, bufs, in_sems, out_sems):
    def start_in(j):
        s = jax.lax.rem(j, _NB)
        pltpu.make_async_copy(
            x_any.at[pl.ds(j * _R, _R)], bufs.at[s], in_sems.at[s]).start()

    def wait_out(j):
        s = jax.lax.rem(j, _NB)
        pltpu.make_async_copy(
            bufs.at[s], copy_any.at[pl.ds(j * _R, _R)], out_sems.at[s]).wait()

    for j in range(_LA):
        start_in(jnp.int32(j))

    def body(i, carry):
        s = jax.lax.rem(i, _NB)
        j = i + _LA

        @pl.when(j < nblk)
        def _():
            @pl.when(j >= _NB)
            def _():
                wait_out(j - _NB)
            start_in(j)

        pltpu.make_async_copy(
            x_any.at[pl.ds(i * _R, _R)], bufs.at[s], in_sems.at[s]).wait()
        pltpu.make_async_copy(
            bufs.at[s], copy_any.at[pl.ds(i * _R, _R)], out_sems.at[s]).start()
        sums_ref[pl.ds(i * _R, _R)] = jnp.sum(bufs[s], axis=(1, 2),
                                              keepdims=True)
        return carry

    jax.lax.fori_loop(0, nblk, body, 0)
    for t in range(_NB):
        wait_out(jnp.int32(nblk - _NB + t))


def _mask_kernel(k, kpad, sums_ref, idx_ref):
    n, c = sums_ref.shape
    hw = jnp.float32(224 * 224)
    xm = sums_ref[...] / hw
    x2_sum = jnp.sum(xm * xm, axis=1, keepdims=True)
    sum_all = jnp.sum(xm, axis=1, keepdims=True)
    neighbor = sum_all - xm
    denom = (1.0 + _B_COEF * x2_sum) * (1.0 + _B_COEF * x2_sum)
    s = neighbor * (xm * xm) / (denom + _EPS)
    ci = jax.lax.broadcasted_iota(jnp.int32, (n, c), 1)
    a = s[:, None, :]
    b = s[:, :, None]
    gt = jnp.sum((a > b).astype(jnp.int32), axis=2)
    eql = jnp.sum(
        ((a == b) & (ci[:, None, :] < ci[:, :, None])).astype(jnp.int32),
        axis=2,
    )
    rank = gt + eql
    jj = jax.lax.broadcasted_iota(jnp.int32, (n, kpad, c), 1)
    hits = (rank[:, None, :] == jj).astype(jnp.int32)
    idx_ref[...] = jnp.sum(hits * ci[:, None, :], axis=2)


def _zero_kernel(nk, idx_ref, x_ref, out_ref, zeros_ref, sem):
    del x_ref
    zeros_ref[...] = jnp.zeros_like(zeros_ref)

    def start(i, _):
        row = idx_ref[i]
        pltpu.make_async_copy(zeros_ref, out_ref.at[pl.ds(row, 1)], sem).start()
        return 0

    jax.lax.fori_loop(0, nk, start, 0)

    def wait(i, _):
        row = idx_ref[i]
        pltpu.make_async_copy(zeros_ref, out_ref.at[pl.ds(row, 1)], sem).wait()
        return 0

    jax.lax.fori_loop(0, nk, wait, 0)


def kernel(x):
    n, c, h, w = x.shape
    rows = n * c
    hw = h * w
    sub = hw // 8
    k = max(1, int(round(_DROP_RATIO * c)))
    kpad = ((k + 7) // 8) * 8
    nblk = rows // _R

    x3 = x.reshape(rows, 8, sub)

    copy, sums = pl.pallas_call(
        lambda x_r, c_r, s_r, bufs, isem, osem: _pass1_kernel(
            nblk, x_r, c_r, s_r, bufs, isem, osem),
        in_specs=[pl.BlockSpec(memory_space=pl.ANY)],
        out_specs=[
            pl.BlockSpec(memory_space=pl.ANY),
            pl.BlockSpec(memory_space=pltpu.VMEM),
        ],
        out_shape=[
            jax.ShapeDtypeStruct((rows, 8, sub), x.dtype),
            jax.ShapeDtypeStruct((rows, 1, 1), jnp.float32),
        ],
        scratch_shapes=[
            pltpu.VMEM((_NB, _R, 8, sub), x.dtype),
            pltpu.SemaphoreType.DMA((_NB,)),
            pltpu.SemaphoreType.DMA((_NB,)),
        ],
    )(x3)

    sums_nc = sums.reshape(n, c)
    idx = pl.pallas_call(
        lambda s_ref, i_ref: _mask_kernel(k, kpad, s_ref, i_ref),
        out_shape=jax.ShapeDtypeStruct((n, kpad), jnp.int32),
    )(sums_nc)
    drop_rows = (idx[:, :k] + jnp.arange(n, dtype=jnp.int32)[:, None] * c
                 ).reshape(n * k)

    out3 = pl.pallas_call(
        lambda i_ref, x_ref, o_ref, z_ref, sem: _zero_kernel(
            n * k, i_ref, x_ref, o_ref, z_ref, sem),
        grid_spec=pltpu.PrefetchScalarGridSpec(
            num_scalar_prefetch=1,
            grid=(1,),
            in_specs=[pl.BlockSpec(memory_space=pl.ANY)],
            out_specs=pl.BlockSpec(memory_space=pl.ANY),
            scratch_shapes=[
                pltpu.VMEM((1, 8, sub), x.dtype),
                pltpu.SemaphoreType.DMA,
            ],
        ),
        out_shape=jax.ShapeDtypeStruct((rows, 8, sub), x.dtype),
        input_output_aliases={1: 0},
    )(drop_rows, copy)

    return out3.reshape(n, c, h, w)


# speedup vs baseline: 2.8342x; 2.8342x over previous
import dataclasses

import jax
import jax.numpy as jnp
from jax.experimental import pallas as pl
from jax.experimental.pallas import tpu as pltpu
from jax.experimental.pallas import tpu_sc as plsc

_DROP_RATIO = 0.2
_B_COEF = 1.0
_C_COEF = 1.0
_EPS = 1e-08

_CB = 8


def _sum_copy_kernel(x_ref, copy_ref, sums_ref):
    blk = x_ref[...]
    copy_ref[...] = blk
    sums_ref[...] = jnp.sum(blk, axis=(2, 3), keepdims=True)


def _sc_mask_kernel(n, c, k, kpad, sums_hbm, lanes_hbm, idx_hbm,
                    srow, lvm, sbuf2, tb, irow, sem):
    nv = c // 16
    core = jax.lax.axis_index("core")
    sub = jax.lax.axis_index("subcore")
    g = sub * 2 + core

    def splat_sum(v):
        for r in (1, 2, 4, 8):
            tb[pl.ds(0, 16)] = v
            tb[pl.ds(16, 16)] = v
            v = v + tb[pl.ds(r, 16)]
        return v

    @pl.when(g < n)
    def _():
        pltpu.async_copy(lanes_hbm, lvm, sem).wait()
        pltpu.async_copy(sums_hbm.at[g], srow, sem).wait()
        lane = lvm[...]
        izero = lane * 0
        ione = izero + 1
        fzero = lane.astype(jnp.float32) * 0.0
        inv_hw = jnp.float32(1.0 / (224.0 * 224.0))
        xm = [srow[pl.ds(16 * j, 16)] * inv_hw for j in range(nv)]
        tot = xm[0]
        for j in range(1, nv):
            tot = tot + xm[j]
        sum_all = splat_sum(tot)
        sq = [v * v for v in xm]
        tot2 = sq[0]
        for j in range(1, nv):
            tot2 = tot2 + sq[j]
        x2_sum = splat_sum(tot2)
        denom = (1.0 + _B_COEF * x2_sum) * (1.0 + _B_COEF * x2_sum)
        scale = denom + _EPS
        s_vecs = [(sum_all - xm[j]) * sq[j] / scale for j in range(nv)]
        for j in range(nv):
            sbuf2[pl.ds(16 * j, 16)] = s_vecs[j]
            sbuf2[pl.ds(c + 16 * j, 16)] = s_vecs[j]
        ranks = [izero for _ in range(nv)]
        for r in range(1, c):
            for j in range(nv):
                w = sbuf2[pl.ds(16 * j + r, 16)]
                gt = w > s_vecs[j]
                wrap = lane >= (c - r - 16 * j)
                eq = (w == s_vecs[j]) & wrap
                ranks[j] = (ranks[j] + jnp.where(gt, ione, izero)
                            + jnp.where(eq, ione, izero))
        cvecs = [lane + 16 * j for j in range(nv)]
        out_vecs = [izero for _ in range(kpad // 16)]
        for s in range(k):
            acc = fzero
            for j in range(nv):
                hits = ranks[j] == s
                acc = acc + jnp.where(hits, cvecs[j].astype(jnp.float32),
                                      fzero)
            chan = splat_sum(acc).astype(jnp.int32)
            t, l = divmod(s, 16)
            out_vecs[t] = out_vecs[t] + jnp.where(lane == l, chan, izero)
        for t in range(kpad // 16):
            irow[pl.ds(16 * t, 16)] = out_vecs[t]
        pltpu.async_copy(irow, idx_hbm.at[g], sem).wait()


def _mask_kernel(k, kpad, sums_ref, idx_ref):
    n, c = sums_ref.shape
    hw = jnp.float32(224 * 224)
    xm = sums_ref[...] / hw
    x2_sum = jnp.sum(xm * xm, axis=1, keepdims=True)
    sum_all = jnp.sum(xm, axis=1, keepdims=True)
    neighbor = sum_all - xm
    denom = (1.0 + _B_COEF * x2_sum) * (1.0 + _B_COEF * x2_sum)
    s = neighbor * (xm * xm) / (denom + _EPS)
    ci = jax.lax.broadcasted_iota(jnp.int32, (n, c), 1)
    a = s[:, None, :]
    b = s[:, :, None]
    gt = jnp.sum((a > b).astype(jnp.int32), axis=2)
    eql = jnp.sum(
        ((a == b) & (ci[:, None, :] < ci[:, :, None])).astype(jnp.int32),
        axis=2,
    )
    rank = gt + eql
    jj = jax.lax.broadcasted_iota(jnp.int32, (n, kpad, c), 1)
    hits = (rank[:, None, :] == jj).astype(jnp.int32)
    idx_ref[...] = jnp.sum(hits * ci[:, None, :], axis=2)


def _zero_kernel(nk, c, idx_ref, x_ref, out_ref, zeros_ref, sem):
    del x_ref
    zeros_ref[...] = jnp.zeros_like(zeros_ref)

    def mk(i):
        row = idx_ref[i]
        nn = jax.lax.div(row, c)
        cc = jax.lax.rem(row, c)
        return pltpu.make_async_copy(
            zeros_ref, out_ref.at[pl.ds(nn, 1), pl.ds(cc, 1)], sem)

    def start(i, _):
        mk(i).start()
        return 0

    jax.lax.fori_loop(0, nk, start, 0)

    def wait(i, _):
        mk(i).wait()
        return 0

    jax.lax.fori_loop(0, nk, wait, 0)


def kernel(x):
    n, c, h, w = x.shape
    k = max(1, int(round(_DROP_RATIO * c)))
    kpad = 32

    copy, sums = pl.pallas_call(
        _sum_copy_kernel,
        grid=(n, c // _CB),
        in_specs=[pl.BlockSpec((1, _CB, h, w), lambda i, j: (i, j, 0, 0))],
        out_specs=[
            pl.BlockSpec((1, _CB, h, w), lambda i, j: (i, j, 0, 0)),
            pl.BlockSpec((1, _CB, 1, 1), lambda i, j: (i, j, 0, 0)),
        ],
        out_shape=[
            jax.ShapeDtypeStruct((n, c, h, w), x.dtype),
            jax.ShapeDtypeStruct((n, c, 1, 1), jnp.float32),
        ],
    )(x)

    sums_nc = sums.reshape(n, c)
    sc_mesh = plsc.VectorSubcoreMesh(core_axis_name="core",
                                     subcore_axis_name="subcore")
    lanes = jnp.arange(16, dtype=jnp.int32)
    idx = pl.kernel(
        lambda s_hbm, l_hbm, i_hbm, srow, lvm, sbuf2, tb, irow, sem:
            _sc_mask_kernel(n, c, k, kpad, s_hbm, l_hbm, i_hbm,
                            srow, lvm, sbuf2, tb, irow, sem),
        out_type=jax.ShapeDtypeStruct((n, kpad), jnp.int32),
        mesh=sc_mesh,
        scratch_types=[
            pltpu.VMEM((c,), jnp.float32),
            pltpu.VMEM((16,), jnp.int32),
            pltpu.VMEM((2 * c,), jnp.float32),
            pltpu.VMEM((32,), jnp.float32),
            pltpu.VMEM((kpad,), jnp.int32),
            pltpu.SemaphoreType.DMA,
        ],
    )(sums_nc, lanes)
    drop_rows = (idx[:, :k] + jnp.arange(n, dtype=jnp.int32)[:, None] * c
                 ).reshape(n * k)

    out = pl.pallas_call(
        lambda i_ref, x_ref, o_ref, z_ref, sem: _zero_kernel(
            n * k, c, i_ref, x_ref, o_ref, z_ref, sem),
        grid_spec=pltpu.PrefetchScalarGridSpec(
            num_scalar_prefetch=1,
            grid=(1,),
            in_specs=[pl.BlockSpec(memory_space=pl.ANY)],
            out_specs=pl.BlockSpec(memory_space=pl.ANY),
            scratch_shapes=[
                pltpu.VMEM((1, 1, h, w), x.dtype),
                pltpu.SemaphoreType.DMA,
            ],
        ),
        out_shape=jax.ShapeDtypeStruct((n, c, h, w), x.dtype),
        input_output_aliases={1: 0},
    )(drop_rows, copy)

    return out
